# baseline (device time: 311723 ns/iter reference)
import jax
import jax.numpy as jnp
from jax import lax
from jax.experimental import pallas as pl
from jax.experimental.pallas import tpu as pltpu

N_DEV = 4
M = 4096
K_SHARD = 1024
N_TOT = 2048
MC = M // N_DEV
NH = N_TOT // 2
SUBS = 8
MCS = MC // SUBS


def kernel(x, w_mat, scale_x, scale_w):
    scale = (scale_x[0] * scale_w[0]).reshape(1, 1).astype(jnp.float32)

    def body(x_ref, w_ref, s_ref, out_ref, comm0, comm1, epi,
             rs_send0, rs_recv0, rs_send1, rs_recv1,
             ag_send0, ag_recv0, ag_send1, ag_recv1,
             epi_sems, credit_sem):
        i = lax.axis_index("i")
        right = lax.rem(i + 1, N_DEV)
        left = lax.rem(i + N_DEV - 1, N_DEV)
        tgt = (right, left)

        barrier = pltpu.get_barrier_semaphore()
        for nbr in (left, right):
            pl.semaphore_signal(barrier, inc=1, device_id=(nbr,),
                                device_id_type=pl.DeviceIdType.MESH)
        pl.semaphore_wait(barrier, 2)

        def mod4(v):
            return lax.rem(v + 2 * N_DEV, N_DEV)

        def pgemm(c, d):
            xa = x_ref[pl.ds(c * MC, MC), :].astype(jnp.bfloat16)
            wb = w_ref[:, d * NH:(d + 1) * NH].astype(jnp.bfloat16)
            return jnp.dot(xa, wb, preferred_element_type=jnp.float32)

        comms = (comm0, comm1)
        rs_send = (rs_send0, rs_send1)
        rs_recv = (rs_recv0, rs_recv1)
        ag_send = (ag_send0, ag_send1)
        ag_recv = (ag_recv0, ag_recv1)

        def rs_rdma(h, d, j):
            rows = pl.ds(j * MCS, MCS)
            return pltpu.make_async_remote_copy(
                src_ref=comms[d].at[h % 2, rows],
                dst_ref=comms[d].at[(h + 1) % 2, rows],
                send_sem=rs_send[d].at[h * SUBS + j],
                recv_sem=rs_recv[d].at[h * SUBS + j],
                device_id=(tgt[d],),
                device_id_type=pl.DeviceIdType.MESH,
            )

        rs_descs = {}
        comm0[0, :, :] = pgemm(mod4(i - 1), 0)
        for j in range(SUBS):
            r = rs_rdma(0, 0, j)
            r.start()
            rs_descs[(0, 0, j)] = r
        comm1[0, :, :] = pgemm(mod4(i + 1), 1)
        for j in range(SUBS):
            r = rs_rdma(0, 1, j)
            r.start()
            rs_descs[(0, 1, j)] = r

        epi[0, :, :] = pgemm(mod4(i - 2), 0)
        epi[1, :, :] = pgemm(mod4(i + 2), 1)
        for j in range(SUBS):
            sl = slice(j * MCS, (j + 1) * MCS)
            for d in range(2):
                rs_descs[(0, d, j)].wait_recv()
                comms[d][1, sl, :] = comms[d][1, sl, :] + epi[d, sl, :]
                r = rs_rdma(1, d, j)
                r.start()
                rs_descs[(1, d, j)] = r

        epi[0, :, :] = pgemm(mod4(i - 3), 0)
        epi[1, :, :] = pgemm(mod4(i + 3), 1)
        for j in range(SUBS):
            sl = slice(j * MCS, (j + 1) * MCS)
            for d in range(2):
                rs_descs[(1, d, j)].wait_recv()
                rs_descs[(0, d, j)].wait_send()
                comms[d][0, sl, :] = comms[d][0, sl, :] + epi[d, sl, :]
        for j in range(SUBS):
            for d in range(2):
                rs_descs[(1, d, j)].wait_send()
        pl.semaphore_signal(credit_sem, inc=1, device_id=(left,),
                            device_id_type=pl.DeviceIdType.MESH)
        pl.semaphore_signal(credit_sem, inc=1, device_id=(right,),
                            device_id_type=pl.DeviceIdType.MESH)
        pl.semaphore_wait(credit_sem, 2)
        for j in range(SUBS):
            for d in range(2):
                r = rs_rdma(2, d, j)
                r.start()
                rs_descs[(2, d, j)] = r

        epi[0, :, :] = pgemm(i, 0)
        epi[1, :, :] = pgemm(i, 1)
        s = s_ref[0, 0]
        copies = {}
        for j in range(SUBS):
            sl = slice(j * MCS, (j + 1) * MCS)
            for d in range(2):
                rs_descs[(2, d, j)].wait_recv()
                y = (comms[d][1, sl, :] + epi[d, sl, :]) * s
                z = jnp.clip(y, -60.0, 60.0)
                epi[d, sl, :] = y / (1.0 + jnp.exp(-z))
                cp = pltpu.make_async_copy(
                    epi.at[d, pl.ds(j * MCS, MCS)],
                    out_ref.at[pl.ds(i * MC + j * MCS, MCS),
                               pl.ds(d * NH, NH)],
                    epi_sems.at[d * SUBS + j],
                )
                cp.start()
                copies[(d, j)] = cp

        def ag_desc(h, d, j, c):
            region = out_ref.at[pl.ds(c * MC + j * MCS, MCS),
                                pl.ds(d * NH, NH)]
            return pltpu.make_async_remote_copy(
                src_ref=region,
                dst_ref=region,
                send_sem=ag_send[d].at[h * SUBS + j],
                recv_sem=ag_recv[d].at[h * SUBS + j],
                device_id=(tgt[d],),
                device_id_type=pl.DeviceIdType.MESH,
            )

        ag_sends = []
        ag_recvs = {}
        for h in range(N_DEV - 1):
            send_c = (mod4(i - h), mod4(i + h))
            recv_c = (mod4(i - 1 - h), mod4(i + 1 + h))
            for j in range(SUBS):
                for d in range(2):
                    if h == 0:
                        sd = pltpu.make_async_remote_copy(
                            src_ref=epi.at[d, pl.ds(j * MCS, MCS)],
                            dst_ref=out_ref.at[pl.ds(i * MC + j * MCS, MCS),
                                               pl.ds(d * NH, NH)],
                            send_sem=ag_send[d].at[j],
                            recv_sem=ag_recv[d].at[j],
                            device_id=(tgt[d],),
                            device_id_type=pl.DeviceIdType.MESH,
                        )
                    else:
                        ag_recvs[(h - 1, d, j)].wait_recv()
                        sd = ag_desc(h, d, j, send_c[d])
                    sd.start()
                    ag_sends.append(sd)
                    ag_recvs[(h, d, j)] = ag_desc(h, d, j, recv_c[d])
        for j in range(SUBS):
            for d in range(2):
                ag_recvs[(N_DEV - 2, d, j)].wait_recv()
        for sd in ag_sends:
            sd.wait_send()
        for cp in copies.values():
            cp.wait()
        for j in range(SUBS):
            for d in range(2):
                rs_descs[(2, d, j)].wait_send()

    return pl.pallas_call(
        body,
        out_shape=jax.ShapeDtypeStruct((M, N_TOT), jnp.float32),
        in_specs=[
            pl.BlockSpec(memory_space=pltpu.VMEM),
            pl.BlockSpec(memory_space=pltpu.VMEM),
            pl.BlockSpec(memory_space=pltpu.SMEM),
        ],
        out_specs=pl.BlockSpec(memory_space=pl.MemorySpace.ANY),
        scratch_shapes=[
            pltpu.VMEM((2, MC, NH), jnp.float32),
            pltpu.VMEM((2, MC, NH), jnp.float32),
            pltpu.VMEM((2, MC, NH), jnp.float32),
            pltpu.SemaphoreType.DMA(((N_DEV - 1) * SUBS,)),
            pltpu.SemaphoreType.DMA(((N_DEV - 1) * SUBS,)),
            pltpu.SemaphoreType.DMA(((N_DEV - 1) * SUBS,)),
            pltpu.SemaphoreType.DMA(((N_DEV - 1) * SUBS,)),
            pltpu.SemaphoreType.DMA(((N_DEV - 1) * SUBS,)),
            pltpu.SemaphoreType.DMA(((N_DEV - 1) * SUBS,)),
            pltpu.SemaphoreType.DMA(((N_DEV - 1) * SUBS,)),
            pltpu.SemaphoreType.DMA(((N_DEV - 1) * SUBS,)),
            pltpu.SemaphoreType.DMA((2 * SUBS,)),
            pltpu.SemaphoreType.REGULAR,
        ],
        compiler_params=pltpu.CompilerParams(
            collective_id=0, vmem_limit_bytes=100 * 1024 * 1024),
    )(x, w_mat, scale)


# device time: 175856 ns/iter; 1.7726x vs baseline; 1.7726x over previous
import jax
import jax.numpy as jnp
from jax import lax
from jax.experimental import pallas as pl
from jax.experimental.pallas import tpu as pltpu

N_DEV = 4
M = 4096
K_SHARD = 1024
N_TOT = 2048
MC = M // N_DEV
NH = N_TOT // 2
SUBS = 8
MCS = MC // SUBS


def kernel(x, w_mat, scale_x, scale_w):
    scale = (scale_x[0] * scale_w[0]).reshape(1, 1).astype(jnp.float32)

    def body(x_ref, w_ref, s_ref, out_ref, comm0, comm1, epi, gath0, gath1,
             fstage, rs_send0, rs_recv0, rs_send1, rs_recv1,
             ag_send0, ag_recv0, ag_send1, ag_recv1,
             out_sems, credit_sem):
        i = lax.axis_index("i")
        right = lax.rem(i + 1, N_DEV)
        left = lax.rem(i + N_DEV - 1, N_DEV)
        tgt = (right, left)

        barrier = pltpu.get_barrier_semaphore()
        for nbr in (left, right):
            pl.semaphore_signal(barrier, inc=1, device_id=(nbr,),
                                device_id_type=pl.DeviceIdType.MESH)
        pl.semaphore_wait(barrier, 2)

        def mod4(v):
            return lax.rem(v + 2 * N_DEV, N_DEV)

        def pgemm(c, d):
            xa = x_ref[pl.ds(c * MC, MC), :].astype(jnp.bfloat16)
            wb = w_ref[:, d * NH:(d + 1) * NH].astype(jnp.bfloat16)
            return jnp.dot(
                xa, wb, preferred_element_type=jnp.float32
            ).astype(jnp.bfloat16)

        comms = (comm0, comm1)
        gaths = (gath0, gath1)
        rs_send = (rs_send0, rs_send1)
        rs_recv = (rs_recv0, rs_recv1)
        ag_send = (ag_send0, ag_send1)
        ag_recv = (ag_recv0, ag_recv1)

        out_copies = {}

        def store_f32(d, vals, row_start, j):
            p = store_f32.parity[d]
            store_f32.parity[d] ^= 1
            if (d, p) in out_copies:
                out_copies[(d, p)].wait()
            fstage[d, p, :, :] = vals
            cp = pltpu.make_async_copy(
                fstage.at[d, p],
                out_ref.at[pl.ds(row_start + j * MCS, MCS),
                           pl.ds(d * NH, NH)],
                out_sems.at[d * 2 + p],
            )
            cp.start()
            out_copies[(d, p)] = cp
        store_f32.parity = [0, 0]

        def rs_rdma(h, d, j):
            rows = pl.ds(j * MCS, MCS)
            return pltpu.make_async_remote_copy(
                src_ref=comms[d].at[h % 2, rows],
                dst_ref=comms[d].at[(h + 1) % 2, rows],
                send_sem=rs_send[d].at[h * SUBS + j],
                recv_sem=rs_recv[d].at[h * SUBS + j],
                device_id=(tgt[d],),
                device_id_type=pl.DeviceIdType.MESH,
            )

        rs_descs = {}
        comm0[0, :, :] = pgemm(mod4(i - 1), 0)
        for j in range(SUBS):
            r = rs_rdma(0, 0, j)
            r.start()
            rs_descs[(0, 0, j)] = r
        comm1[0, :, :] = pgemm(mod4(i + 1), 1)
        for j in range(SUBS):
            r = rs_rdma(0, 1, j)
            r.start()
            rs_descs[(0, 1, j)] = r

        epi[0, :, :] = pgemm(mod4(i - 2), 0)
        epi[1, :, :] = pgemm(mod4(i + 2), 1)
        for j in range(SUBS):
            sl = slice(j * MCS, (j + 1) * MCS)
            for d in range(2):
                rs_descs[(0, d, j)].wait_recv()
                comms[d][1, sl, :] = comms[d][1, sl, :] + epi[d, sl, :]
                r = rs_rdma(1, d, j)
                r.start()
                rs_descs[(1, d, j)] = r

        epi[0, :, :] = pgemm(mod4(i - 3), 0)
        epi[1, :, :] = pgemm(mod4(i + 3), 1)
        for j in range(SUBS):
            sl = slice(j * MCS, (j + 1) * MCS)
            for d in range(2):
                rs_descs[(1, d, j)].wait_recv()
                rs_descs[(0, d, j)].wait_send()
                comms[d][0, sl, :] = comms[d][0, sl, :] + epi[d, sl, :]
        for j in range(SUBS):
            for d in range(2):
                rs_descs[(1, d, j)].wait_send()
        pl.semaphore_signal(credit_sem, inc=1, device_id=(left,),
                            device_id_type=pl.DeviceIdType.MESH)
        pl.semaphore_signal(credit_sem, inc=1, device_id=(right,),
                            device_id_type=pl.DeviceIdType.MESH)
        pl.semaphore_wait(credit_sem, 2)
        for j in range(SUBS):
            for d in range(2):
                r = rs_rdma(2, d, j)
                r.start()
                rs_descs[(2, d, j)] = r

        epi[0, :, :] = pgemm(i, 0)
        epi[1, :, :] = pgemm(i, 1)
        s = s_ref[0, 0]

        def ag_rdma(h, d, j):
            rows = pl.ds(j * MCS, MCS)
            src = epi.at[d, rows] if h == 0 else gaths[d].at[h - 1, rows]
            return pltpu.make_async_remote_copy(
                src_ref=src,
                dst_ref=gaths[d].at[h, rows],
                send_sem=ag_send[d].at[h * SUBS + j],
                recv_sem=ag_recv[d].at[h * SUBS + j],
                device_id=(tgt[d],),
                device_id_type=pl.DeviceIdType.MESH,
            )

        ag_descs = {}
        for j in range(SUBS):
            sl = slice(j * MCS, (j + 1) * MCS)
            for d in range(2):
                rs_descs[(2, d, j)].wait_recv()
                y = (comms[d][1, sl, :].astype(jnp.float32)
                     + epi[d, sl, :].astype(jnp.float32)) * s
                z = jnp.clip(y, -60.0, 60.0)
                res = y / (1.0 + jnp.exp(-z))
                epi[d, sl, :] = res.astype(jnp.bfloat16)
                r = ag_rdma(0, d, j)
                r.start()
                ag_descs[(0, d, j)] = r
                store_f32(d, res, i * MC, j)

        for h in range(N_DEV - 1):
            recv_c = (mod4(i - 1 - h), mod4(i + 1 + h))
            for j in range(SUBS):
                sl = slice(j * MCS, (j + 1) * MCS)
                for d in range(2):
                    ag_descs[(h, d, j)].wait_recv()
                    if h < N_DEV - 2:
                        r = ag_rdma(h + 1, d, j)
                        r.start()
                        ag_descs[(h + 1, d, j)] = r
                    store_f32(d, gaths[d][h, sl, :].astype(jnp.float32),
                              recv_c[d] * MC, j)

        for h in range(N_DEV - 1):
            for j in range(SUBS):
                for d in range(2):
                    ag_descs[(h, d, j)].wait_send()
        for j in range(SUBS):
            for d in range(2):
                rs_descs[(2, d, j)].wait_send()
        for cp in out_copies.values():
            cp.wait()

    return pl.pallas_call(
        body,
        out_shape=jax.ShapeDtypeStruct((M, N_TOT), jnp.float32),
        in_specs=[
            pl.BlockSpec(memory_space=pltpu.VMEM),
            pl.BlockSpec(memory_space=pltpu.VMEM),
            pl.BlockSpec(memory_space=pltpu.SMEM),
        ],
        out_specs=pl.BlockSpec(memory_space=pl.MemorySpace.ANY),
        scratch_shapes=[
            pltpu.VMEM((2, MC, NH), jnp.bfloat16),
            pltpu.VMEM((2, MC, NH), jnp.bfloat16),
            pltpu.VMEM((2, MC, NH), jnp.bfloat16),
            pltpu.VMEM((N_DEV - 1, MC, NH), jnp.bfloat16),
            pltpu.VMEM((N_DEV - 1, MC, NH), jnp.bfloat16),
            pltpu.VMEM((2, 2, MCS, NH), jnp.float32),
            pltpu.SemaphoreType.DMA(((N_DEV - 1) * SUBS,)),
            pltpu.SemaphoreType.DMA(((N_DEV - 1) * SUBS,)),
            pltpu.SemaphoreType.DMA(((N_DEV - 1) * SUBS,)),
            pltpu.SemaphoreType.DMA(((N_DEV - 1) * SUBS,)),
            pltpu.SemaphoreType.DMA(((N_DEV - 1) * SUBS,)),
            pltpu.SemaphoreType.DMA(((N_DEV - 1) * SUBS,)),
            pltpu.SemaphoreType.DMA(((N_DEV - 1) * SUBS,)),
            pltpu.SemaphoreType.DMA(((N_DEV - 1) * SUBS,)),
            pltpu.SemaphoreType.DMA((4,)),
            pltpu.SemaphoreType.REGULAR,
        ],
        compiler_params=pltpu.CompilerParams(
            collective_id=0, vmem_limit_bytes=100 * 1024 * 1024),
    )(x, w_mat, scale)
